# baseline (speedup 1.0000x reference)
import dataclasses

import jax
import jax.numpy as jnp
from jax import lax
from jax.experimental import pallas as pl
from jax.experimental.pallas import tpu as pltpu
from jax.experimental.pallas import tpu_sc as plsc

_N = 10000
_E = 320000
_D = 128
_EPS = 1e-5
_CHUNK = 128
_NCHUNKS = _E // _CHUNK
_NTILES = 32
_ZP = 80
_DEG_W = 16

_TCH = 80
_HCH = _TCH // 2
_EPAD = _NTILES * _TCH * _CHUNK
_PAD = 80
_NACC = _N + _PAD
_ZPIECES = _NACC // _ZP
_DPIECES = _N // _ZP
_ZITERS = (_ZPIECES + 15) // 16
_DITERS = (_DPIECES + 15) // 16


def _sc_compiler_params():
    cp = pltpu.CompilerParams()
    if "needs_layout_passes" in pltpu.CompilerParams.__dataclass_fields__:
        cp = dataclasses.replace(cp, needs_layout_passes=False)
    return cp

def _sc_degree_body(col_hbm, out_hbm, counts, col_v):
    cid = lax.axis_index("c")
    sid = lax.axis_index("s")
    w = sid * 2 + cid

    zero16 = jnp.zeros((16,), jnp.float32)
    one16 = jnp.ones((16,), jnp.float32)

    @pl.loop(0, _N // 16)
    def _(i):
        counts[pl.ds(i * 16, 16)] = zero16

    @pl.loop(0, (_NCHUNKS + _NTILES - 1) // _NTILES)
    def _(kk):
        ch = kk * _NTILES + w

        @pl.when(ch < _NCHUNKS)
        def _():
            pltpu.sync_copy(col_hbm.at[pl.ds(ch * _CHUNK, _CHUNK)], col_v)
            for j in range(_CHUNK // 16):
                idx = col_v[pl.ds(j * 16, 16)]
                plsc.addupdate_scatter(counts, [idx], one16)

    pltpu.sync_copy(counts, out_hbm.at[w, 0])


def _sc_degree(col1d):
    mesh = plsc.VectorSubcoreMesh(core_axis_name="c", subcore_axis_name="s")
    kern = pl.kernel(
        _sc_degree_body,
        out_type=jax.ShapeDtypeStruct((_NTILES, 1, _N), jnp.float32),
        mesh=mesh,
        scratch_types=[
            pltpu.VMEM((_N,), jnp.float32),
            pltpu.VMEM((_CHUNK,), jnp.int32),
        ],
        compiler_params=_sc_compiler_params(),
    )
    return kern(col1d)


def _sc_propagate_body(h_hbm, row_hbm, col_hbm, out_hbm, acc, row_slab,
                       col_slab, msgs, gsems):
    cid = lax.axis_index("c")
    sid = lax.axis_index("s")
    w = sid * 2 + cid

    zero16 = jnp.zeros((16,), jnp.float32)

    @pl.loop(0, _ZP)
    def _(rr):
        for jj in range(_D // 16):
            msgs[0, rr, pl.ds(jj * 16, 16)] = zero16

    @pl.loop(0, _ZITERS)
    def _(kk):
        p = kk * 16 + sid

        @pl.when(p < _ZPIECES)
        def _():
            pltpu.sync_copy(msgs.at[0, pl.ds(0, _ZP)],
                            acc.at[pl.ds(p * _ZP, _ZP)])

    plsc.subcore_barrier()

    for half in range(2):
        pltpu.sync_copy(row_hbm.at[w, pl.ds(half * _HCH, _HCH)], row_slab)
        pltpu.sync_copy(col_hbm.at[w, pl.ds(half * _HCH, _HCH)], col_slab)

        @pl.loop(0, _HCH // 2)
        def _(k):
            j = 2 * k
            cp0 = pltpu.async_copy(h_hbm.at[row_slab.at[j]], msgs.at[0],
                                   gsems.at[0])
            cp1 = pltpu.async_copy(h_hbm.at[row_slab.at[j + 1]], msgs.at[1],
                                   gsems.at[1])
            cp0.wait()
            pltpu.sync_copy(msgs.at[0], acc.at[col_slab.at[j]], add=True)
            cp1.wait()
            pltpu.sync_copy(msgs.at[1], acc.at[col_slab.at[j + 1]], add=True)

    plsc.subcore_barrier()

    @pl.loop(0, _DITERS)
    def _(kk):
        p = kk * 16 + sid

        @pl.when(p < _DPIECES)
        def _():
            pltpu.sync_copy(acc.at[pl.ds(p * _ZP, _ZP)],
                            out_hbm.at[cid, pl.ds(p * _ZP, _ZP)])


def _sc_propagate(h, row3d, col3d):
    mesh = plsc.VectorSubcoreMesh(core_axis_name="c", subcore_axis_name="s")
    kern = pl.kernel(
        _sc_propagate_body,
        out_type=jax.ShapeDtypeStruct((2, _N, _D), jnp.float32),
        mesh=mesh,
        scratch_types=[
            pltpu.VMEM_SHARED((_NACC, _D), jnp.float32),
            pltpu.VMEM((_HCH, _CHUNK), jnp.int32),
            pltpu.VMEM((_HCH, _CHUNK), jnp.int32),
            pltpu.VMEM((2, _CHUNK, _D), jnp.float32),
            pltpu.SemaphoreType.DMA((2,)),
        ],
        compiler_params=_sc_compiler_params(),
    )
    return kern(h, row3d, col3d)


def _matmul_t(a, w_ref):
    return lax.dot_general(a, w_ref[...], (((1,), (1,)), ((), ())),
                           preferred_element_type=jnp.float32,
                           precision=lax.Precision.HIGHEST)


def _bn(h, g, b):
    m = jnp.mean(h, axis=0, keepdims=True)
    v = jnp.mean((h - m) ** 2, axis=0, keepdims=True)
    return (h - m) / jnp.sqrt(v + _EPS) * g + b


def _rsqrt_deg(degp):
    dp = degp[:, 0, :]
    ones = jnp.ones((_NTILES, 1), jnp.float32)
    deg = lax.dot_general(dp, ones, (((0,), (0,)), ((), ())),
                          preferred_element_type=jnp.float32,
                          precision=lax.Precision.HIGHEST)
    return jnp.where(deg > 0.0, lax.rsqrt(deg), 0.0)


def _stage_a_body(x_ref, w_ref, b_ref, g_ref, bb_ref, o_ref):
    h = _matmul_t(x_ref[...], w_ref) + b_ref[...]
    h = _bn(h, g_ref[...], bb_ref[...])
    o_ref[...] = jnp.maximum(h, 0.0)


def _scale_body(h_ref, degp_ref, o_ref):
    r = _rsqrt_deg(degp_ref[...])
    o_ref[...] = h_ref[...] * r


def _stage_c_body(sp_ref, degp_ref, h0_ref, w_ref, b_ref, g_ref, bb_ref,
                  o_ref):
    r = _rsqrt_deg(degp_ref[...])
    sp = sp_ref[...]
    s = (sp[0] + sp[1]) * r
    h = _matmul_t(s, w_ref) + b_ref[...]
    h = jnp.maximum(_bn(h, g_ref[...], bb_ref[...]), 0.0) + h0_ref[...]
    o_ref[...] = h * r


def _stage_d_body(sp_ref, degp_ref, h0_ref, w_ref, b_ref, g_ref, bb_ref,
                  wc_ref, bc_ref, o_ref):
    r = _rsqrt_deg(degp_ref[...])
    sp = sp_ref[...]
    s = (sp[0] + sp[1]) * r
    h = _matmul_t(s, w_ref) + b_ref[...]
    h = jnp.maximum(_bn(h, g_ref[...], bb_ref[...]), 0.0) + h0_ref[...]
    o_ref[...] = _matmul_t(h, wc_ref) + bc_ref[...]


def _f32_out(shape):
    return jax.ShapeDtypeStruct(shape, jnp.float32)


def kernel(x, edge_index, fc0_W, fc0_b, conv1_W, conv1_b, conv2_W, conv2_b,
           bn0_g, bn0_b, bn1_g, bn1_b, bn2_g, bn2_b, cls_W, cls_b):
    ei = edge_index.astype(jnp.int32)
    row1d = ei[0]
    col1d = ei[1]
    npad = _EPAD - _E
    row3d = jnp.concatenate(
        [row1d, jnp.zeros((npad,), jnp.int32)]).reshape(
            _TCH, _NTILES, _CHUNK).transpose(1, 0, 2)
    col3d = jnp.concatenate(
        [col1d, _N + (jnp.arange(npad, dtype=jnp.int32) % _PAD)]).reshape(
            _TCH, _NTILES, _CHUNK).transpose(1, 0, 2)

    b0 = fc0_b.reshape(1, _D)
    g0 = bn0_g.reshape(1, _D)
    bb0 = bn0_b.reshape(1, _D)
    b1 = conv1_b.reshape(1, _D)
    g1 = bn1_g.reshape(1, _D)
    bb1 = bn1_b.reshape(1, _D)
    b2 = conv2_b.reshape(1, _D)
    g2 = bn2_g.reshape(1, _D)
    bb2 = bn2_b.reshape(1, _D)
    bc = cls_b.reshape(1, _D)

    degp = _sc_degree(col1d)

    h0 = pl.pallas_call(_stage_a_body, out_shape=_f32_out((_N, _D)))(
        x, fc0_W, b0, g0, bb0)

    h0s = pl.pallas_call(_scale_body, out_shape=_f32_out((_N, _D)))(
        h0, degp)

    s1p = _sc_propagate(h0s, row3d, col3d)

    h1s = pl.pallas_call(_stage_c_body, out_shape=_f32_out((_N, _D)))(
        s1p, degp, h0, conv1_W, b1, g1, bb1)

    s2p = _sc_propagate(h1s, row3d, col3d)

    out = pl.pallas_call(_stage_d_body, out_shape=_f32_out((_N, _D)))(
        s2p, degp, h0, conv2_W, b2, g2, bb2, cls_W, bc)

    return out


# speedup vs baseline: 2.0148x; 2.0148x over previous
import dataclasses

import jax
import jax.numpy as jnp
from jax import lax
from jax.experimental import pallas as pl
from jax.experimental.pallas import tpu as pltpu
from jax.experimental.pallas import tpu_sc as plsc

_N = 10000
_E = 320000
_D = 128
_EPS = 1e-5
_CHUNK = 128
_NCHUNKS = _E // _CHUNK
_NTILES = 32
_ZP = 80
_DEG_W = 16

_TCH = 80
_HCH = _TCH // 2
_EPAD = _NTILES * _TCH * _CHUNK
_PAD = 80
_NACC = _N + _PAD
_ZPIECES = _NACC // _ZP
_DPIECES = _N // _ZP
_ZITERS = (_ZPIECES + 15) // 16
_DITERS = (_DPIECES + 15) // 16


def _sc_compiler_params():
    cp = pltpu.CompilerParams()
    if "needs_layout_passes" in pltpu.CompilerParams.__dataclass_fields__:
        cp = dataclasses.replace(cp, needs_layout_passes=False)
    return cp

def _sc_degree_body(col_hbm, out_hbm, counts, col_v):
    cid = lax.axis_index("c")
    sid = lax.axis_index("s")
    w = sid * 2 + cid

    zero16 = jnp.zeros((16,), jnp.float32)
    one16 = jnp.ones((16,), jnp.float32)

    @pl.loop(0, _N // 16)
    def _(i):
        counts[pl.ds(i * 16, 16)] = zero16

    @pl.loop(0, (_NCHUNKS + _NTILES - 1) // _NTILES)
    def _(kk):
        ch = kk * _NTILES + w

        @pl.when(ch < _NCHUNKS)
        def _():
            pltpu.sync_copy(col_hbm.at[pl.ds(ch * _CHUNK, _CHUNK)], col_v)
            for j in range(_CHUNK // 16):
                idx = col_v[pl.ds(j * 16, 16)]
                plsc.addupdate_scatter(counts, [idx], one16)

    pltpu.sync_copy(counts, out_hbm.at[w, 0])


def _sc_degree(col1d):
    mesh = plsc.VectorSubcoreMesh(core_axis_name="c", subcore_axis_name="s")
    kern = pl.kernel(
        _sc_degree_body,
        out_type=jax.ShapeDtypeStruct((_NTILES, 1, _N), jnp.float32),
        mesh=mesh,
        scratch_types=[
            pltpu.VMEM((_N,), jnp.float32),
            pltpu.VMEM((_CHUNK,), jnp.int32),
        ],
        compiler_params=_sc_compiler_params(),
    )
    return kern(col1d)


def _sc_propagate_body(h_hbm, row_hbm, col_hbm, out_hbm, acc, row_v0,
                       col_v0, row_v1, col_v1, msgs, gsems):
    cid = lax.axis_index("c")
    sid = lax.axis_index("s")
    w = sid * 2 + cid

    zero16 = jnp.zeros((16,), jnp.float32)

    @pl.loop(0, _ZP)
    def _(rr):
        for jj in range(_D // 16):
            msgs[0, rr, pl.ds(jj * 16, 16)] = zero16

    @pl.loop(0, _ZITERS)
    def _(kk):
        p = kk * 16 + sid

        @pl.when(p < _ZPIECES)
        def _():
            pltpu.sync_copy(msgs.at[0, pl.ds(0, _ZP)],
                            acc.at[pl.ds(p * _ZP, _ZP)])

    plsc.subcore_barrier()

    def _one(ch, row_v, col_v, b):
        pltpu.sync_copy(row_hbm.at[pl.ds(ch * _CHUNK, _CHUNK)], row_v)
        pltpu.sync_copy(col_hbm.at[pl.ds(ch * _CHUNK, _CHUNK)], col_v)
        return pltpu.async_copy(h_hbm.at[row_v], msgs.at[b], gsems.at[b])

    @pl.loop(0, _NCHUNKS // _NTILES // 2)
    def _(k2):
        ch0 = (2 * k2) * _NTILES + w
        ch1 = ch0 + _NTILES
        cp0 = _one(ch0, row_v0, col_v0, 0)
        cp1 = _one(ch1, row_v1, col_v1, 1)
        cp0.wait()
        pltpu.sync_copy(msgs.at[0], acc.at[col_v0], add=True)
        cp1.wait()
        pltpu.sync_copy(msgs.at[1], acc.at[col_v1], add=True)

    @pl.when(w < _NCHUNKS - (_NCHUNKS // _NTILES) * _NTILES)
    def _():
        ch = (_NCHUNKS // _NTILES) * _NTILES + w
        cp = _one(ch, row_v0, col_v0, 0)
        cp.wait()
        pltpu.sync_copy(msgs.at[0], acc.at[col_v0], add=True)

    plsc.subcore_barrier()

    @pl.loop(0, _DITERS)
    def _(kk):
        p = kk * 16 + sid

        @pl.when(p < _DPIECES)
        def _():
            pltpu.sync_copy(acc.at[pl.ds(p * _ZP, _ZP)],
                            out_hbm.at[cid, pl.ds(p * _ZP, _ZP)])


def _sc_propagate(h, row1d, col1d):
    mesh = plsc.VectorSubcoreMesh(core_axis_name="c", subcore_axis_name="s")
    kern = pl.kernel(
        _sc_propagate_body,
        out_type=jax.ShapeDtypeStruct((2, _N, _D), jnp.float32),
        mesh=mesh,
        scratch_types=[
            pltpu.VMEM_SHARED((_NACC, _D), jnp.float32),
            pltpu.VMEM((_CHUNK,), jnp.int32),
            pltpu.VMEM((_CHUNK,), jnp.int32),
            pltpu.VMEM((_CHUNK,), jnp.int32),
            pltpu.VMEM((_CHUNK,), jnp.int32),
            pltpu.VMEM((2, _CHUNK, _D), jnp.float32),
            pltpu.SemaphoreType.DMA((2,)),
        ],
        compiler_params=_sc_compiler_params(),
    )
    return kern(h, row1d, col1d)


def _matmul_t(a, w_ref):
    return lax.dot_general(a, w_ref[...], (((1,), (1,)), ((), ())),
                           preferred_element_type=jnp.float32,
                           precision=lax.Precision.HIGHEST)


def _bn(h, g, b):
    m = jnp.mean(h, axis=0, keepdims=True)
    v = jnp.mean((h - m) ** 2, axis=0, keepdims=True)
    return (h - m) / jnp.sqrt(v + _EPS) * g + b


def _rsqrt_deg(degp):
    dp = degp[:, 0, :]
    ones = jnp.ones((_NTILES, 1), jnp.float32)
    deg = lax.dot_general(dp, ones, (((0,), (0,)), ((), ())),
                          preferred_element_type=jnp.float32,
                          precision=lax.Precision.HIGHEST)
    return jnp.where(deg > 0.0, lax.rsqrt(deg), 0.0)


def _stage_a_body(x_ref, w_ref, b_ref, g_ref, bb_ref, o_ref):
    h = _matmul_t(x_ref[...], w_ref) + b_ref[...]
    h = _bn(h, g_ref[...], bb_ref[...])
    o_ref[...] = jnp.maximum(h, 0.0)


def _scale_body(h_ref, degp_ref, o_ref):
    r = _rsqrt_deg(degp_ref[...])
    o_ref[...] = h_ref[...] * r


def _stage_c_body(sp_ref, degp_ref, h0_ref, w_ref, b_ref, g_ref, bb_ref,
                  o_ref):
    r = _rsqrt_deg(degp_ref[...])
    sp = sp_ref[...]
    s = (sp[0] + sp[1]) * r
    h = _matmul_t(s, w_ref) + b_ref[...]
    h = jnp.maximum(_bn(h, g_ref[...], bb_ref[...]), 0.0) + h0_ref[...]
    o_ref[...] = h * r


def _stage_d_body(sp_ref, degp_ref, h0_ref, w_ref, b_ref, g_ref, bb_ref,
                  wc_ref, bc_ref, o_ref):
    r = _rsqrt_deg(degp_ref[...])
    sp = sp_ref[...]
    s = (sp[0] + sp[1]) * r
    h = _matmul_t(s, w_ref) + b_ref[...]
    h = jnp.maximum(_bn(h, g_ref[...], bb_ref[...]), 0.0) + h0_ref[...]
    o_ref[...] = _matmul_t(h, wc_ref) + bc_ref[...]


def _f32_out(shape):
    return jax.ShapeDtypeStruct(shape, jnp.float32)


def kernel(x, edge_index, fc0_W, fc0_b, conv1_W, conv1_b, conv2_W, conv2_b,
           bn0_g, bn0_b, bn1_g, bn1_b, bn2_g, bn2_b, cls_W, cls_b):
    ei = edge_index.astype(jnp.int32)
    row1d = ei[0]
    col1d = ei[1]

    b0 = fc0_b.reshape(1, _D)
    g0 = bn0_g.reshape(1, _D)
    bb0 = bn0_b.reshape(1, _D)
    b1 = conv1_b.reshape(1, _D)
    g1 = bn1_g.reshape(1, _D)
    bb1 = bn1_b.reshape(1, _D)
    b2 = conv2_b.reshape(1, _D)
    g2 = bn2_g.reshape(1, _D)
    bb2 = bn2_b.reshape(1, _D)
    bc = cls_b.reshape(1, _D)

    degp = _sc_degree(col1d)

    h0 = pl.pallas_call(_stage_a_body, out_shape=_f32_out((_N, _D)))(
        x, fc0_W, b0, g0, bb0)

    h0s = pl.pallas_call(_scale_body, out_shape=_f32_out((_N, _D)))(
        h0, degp)

    s1p = _sc_propagate(h0s, row1d, col1d)

    h1s = pl.pallas_call(_stage_c_body, out_shape=_f32_out((_N, _D)))(
        s1p, degp, h0, conv1_W, b1, g1, bb1)

    s2p = _sc_propagate(h1s, row1d, col1d)

    out = pl.pallas_call(_stage_d_body, out_shape=_f32_out((_N, _D)))(
        s2p, degp, h0, conv2_W, b2, g2, bb2, cls_W, bc)

    return out


# speedup vs baseline: 2.4286x; 1.2054x over previous
import dataclasses

import jax
import jax.numpy as jnp
from jax import lax
from jax.experimental import pallas as pl
from jax.experimental.pallas import tpu as pltpu
from jax.experimental.pallas import tpu_sc as plsc

_N = 10000
_E = 320000
_D = 128
_EPS = 1e-5
_CHUNK = 128
_NCHUNKS = _E // _CHUNK
_NTILES = 32
_ZP = 80
_DEG_W = 16

_NACC = _N
_ZPIECES = _NACC // _ZP
_DPIECES = _N // _ZP
_ZITERS = (_ZPIECES + 15) // 16
_DITERS = (_DPIECES + 15) // 16
_UNROLL = 3


def _sc_compiler_params():
    cp = pltpu.CompilerParams()
    if "needs_layout_passes" in pltpu.CompilerParams.__dataclass_fields__:
        cp = dataclasses.replace(cp, needs_layout_passes=False)
    return cp

def _sc_degree_body(col_hbm, out_hbm, counts, col_v):
    cid = lax.axis_index("c")
    sid = lax.axis_index("s")
    w = sid * 2 + cid

    zero16 = jnp.zeros((16,), jnp.float32)
    one16 = jnp.ones((16,), jnp.float32)

    @pl.loop(0, _N // 16)
    def _(i):
        counts[pl.ds(i * 16, 16)] = zero16

    @pl.loop(0, (_NCHUNKS + _NTILES - 1) // _NTILES)
    def _(kk):
        ch = kk * _NTILES + w

        @pl.when(ch < _NCHUNKS)
        def _():
            pltpu.sync_copy(col_hbm.at[pl.ds(ch * _CHUNK, _CHUNK)], col_v)
            for j in range(_CHUNK // 16):
                idx = col_v[pl.ds(j * 16, 16)]
                plsc.addupdate_scatter(counts, [idx], one16)

    pltpu.sync_copy(counts, out_hbm.at[w, 0])


def _sc_degree(col1d):
    mesh = plsc.VectorSubcoreMesh(core_axis_name="c", subcore_axis_name="s")
    kern = pl.kernel(
        _sc_degree_body,
        out_type=jax.ShapeDtypeStruct((_NTILES, 1, _N), jnp.float32),
        mesh=mesh,
        scratch_types=[
            pltpu.VMEM((_N,), jnp.float32),
            pltpu.VMEM((_CHUNK,), jnp.int32),
        ],
        compiler_params=_sc_compiler_params(),
    )
    return kern(col1d)


def _sc_propagate_body(h_hbm, rc_hbm, out_hbm, acc, rc_v, msgs, gsems):
    cid = lax.axis_index("c")
    sid = lax.axis_index("s")
    w = sid * 2 + cid

    zero16 = jnp.zeros((16,), jnp.float32)

    @pl.loop(0, _ZP)
    def _(rr):
        for jj in range(_D // 16):
            msgs[0, rr, pl.ds(jj * 16, 16)] = zero16

    @pl.loop(0, _ZITERS)
    def _(kk):
        p = kk * 16 + sid

        @pl.when(p < _ZPIECES)
        def _():
            pltpu.sync_copy(msgs.at[0, pl.ds(0, _ZP)],
                            acc.at[pl.ds(p * _ZP, _ZP)])

    plsc.subcore_barrier()

    def _one(ch, b):
        pltpu.sync_copy(rc_hbm.at[ch], rc_v.at[b])
        return pltpu.async_copy(h_hbm.at[rc_v.at[b, 0]], msgs.at[b],
                                gsems.at[b])

    def _flush(b):
        pltpu.sync_copy(msgs.at[b], acc.at[rc_v.at[b, 1]], add=True)

    @pl.loop(0, _NCHUNKS // _NTILES // _UNROLL)
    def _(kg):
        base = (_UNROLL * kg) * _NTILES + w
        cps = [_one(base + b * _NTILES, b) for b in range(_UNROLL)]
        for b in range(_UNROLL):
            cps[b].wait()
            _flush(b)

    @pl.when(w < _NCHUNKS - (_NCHUNKS // _NTILES) * _NTILES)
    def _():
        ch = (_NCHUNKS // _NTILES) * _NTILES + w
        cp = _one(ch, 0)
        cp.wait()
        _flush(0)

    plsc.subcore_barrier()

    @pl.loop(0, _DITERS)
    def _(kk):
        p = kk * 16 + sid

        @pl.when(p < _DPIECES)
        def _():
            pltpu.sync_copy(acc.at[pl.ds(p * _ZP, _ZP)],
                            out_hbm.at[cid, pl.ds(p * _ZP, _ZP)])


def _sc_propagate(h, rc3d):
    mesh = plsc.VectorSubcoreMesh(core_axis_name="c", subcore_axis_name="s")
    kern = pl.kernel(
        _sc_propagate_body,
        out_type=jax.ShapeDtypeStruct((2, _N, _D), jnp.float32),
        mesh=mesh,
        scratch_types=[
            pltpu.VMEM_SHARED((_NACC, _D), jnp.float32),
            pltpu.VMEM((_UNROLL, 2, _CHUNK), jnp.int32),
            pltpu.VMEM((_UNROLL, _CHUNK, _D), jnp.float32),
            pltpu.SemaphoreType.DMA((_UNROLL,)),
        ],
        compiler_params=_sc_compiler_params(),
    )
    return kern(h, rc3d)


def _matmul_t(a, w_ref):
    return lax.dot_general(a, w_ref[...], (((1,), (1,)), ((), ())),
                           preferred_element_type=jnp.float32,
                           precision=lax.Precision.HIGHEST)


def _bn(h, g, b):
    m = jnp.mean(h, axis=0, keepdims=True)
    v = jnp.mean((h - m) ** 2, axis=0, keepdims=True)
    return (h - m) / jnp.sqrt(v + _EPS) * g + b


def _rsqrt_deg(degp):
    dp = degp[:, 0, :]
    ones = jnp.ones((_NTILES, 1), jnp.float32)
    deg = lax.dot_general(dp, ones, (((0,), (0,)), ((), ())),
                          preferred_element_type=jnp.float32,
                          precision=lax.Precision.HIGHEST)
    return jnp.where(deg > 0.0, lax.rsqrt(deg), 0.0)


def _stage_a_body(x_ref, w_ref, b_ref, g_ref, bb_ref, o_ref):
    h = _matmul_t(x_ref[...], w_ref) + b_ref[...]
    h = _bn(h, g_ref[...], bb_ref[...])
    o_ref[...] = jnp.maximum(h, 0.0)


def _scale_body(h_ref, degp_ref, o_ref):
    r = _rsqrt_deg(degp_ref[...])
    o_ref[...] = h_ref[...] * r


def _stage_c_body(sp_ref, degp_ref, h0_ref, w_ref, b_ref, g_ref, bb_ref,
                  o_ref):
    r = _rsqrt_deg(degp_ref[...])
    sp = sp_ref[...]
    s = (sp[0] + sp[1]) * r
    h = _matmul_t(s, w_ref) + b_ref[...]
    h = jnp.maximum(_bn(h, g_ref[...], bb_ref[...]), 0.0) + h0_ref[...]
    o_ref[...] = h * r


def _stage_d_body(sp_ref, degp_ref, h0_ref, w_ref, b_ref, g_ref, bb_ref,
                  wc_ref, bc_ref, o_ref):
    r = _rsqrt_deg(degp_ref[...])
    sp = sp_ref[...]
    s = (sp[0] + sp[1]) * r
    h = _matmul_t(s, w_ref) + b_ref[...]
    h = jnp.maximum(_bn(h, g_ref[...], bb_ref[...]), 0.0) + h0_ref[...]
    o_ref[...] = _matmul_t(h, wc_ref) + bc_ref[...]


def _f32_out(shape):
    return jax.ShapeDtypeStruct(shape, jnp.float32)


def kernel(x, edge_index, fc0_W, fc0_b, conv1_W, conv1_b, conv2_W, conv2_b,
           bn0_g, bn0_b, bn1_g, bn1_b, bn2_g, bn2_b, cls_W, cls_b):
    ei = edge_index.astype(jnp.int32)
    col1d = ei[1]
    rc3d = ei.reshape(2, _NCHUNKS, _CHUNK).transpose(1, 0, 2)

    b0 = fc0_b.reshape(1, _D)
    g0 = bn0_g.reshape(1, _D)
    bb0 = bn0_b.reshape(1, _D)
    b1 = conv1_b.reshape(1, _D)
    g1 = bn1_g.reshape(1, _D)
    bb1 = bn1_b.reshape(1, _D)
    b2 = conv2_b.reshape(1, _D)
    g2 = bn2_g.reshape(1, _D)
    bb2 = bn2_b.reshape(1, _D)
    bc = cls_b.reshape(1, _D)

    degp = _sc_degree(col1d)

    h0 = pl.pallas_call(_stage_a_body, out_shape=_f32_out((_N, _D)))(
        x, fc0_W, b0, g0, bb0)

    h0s = pl.pallas_call(_scale_body, out_shape=_f32_out((_N, _D)))(
        h0, degp)

    s1p = _sc_propagate(h0s, rc3d)

    h1s = pl.pallas_call(_stage_c_body, out_shape=_f32_out((_N, _D)))(
        s1p, degp, h0, conv1_W, b1, g1, bb1)

    s2p = _sc_propagate(h1s, rc3d)

    out = pl.pallas_call(_stage_d_body, out_shape=_f32_out((_N, _D)))(
        s2p, degp, h0, conv2_W, b2, g2, bb2, cls_W, bc)

    return out


# speedup vs baseline: 2.4475x; 1.0078x over previous
import dataclasses

import jax
import jax.numpy as jnp
from jax import lax
from jax.experimental import pallas as pl
from jax.experimental.pallas import tpu as pltpu
from jax.experimental.pallas import tpu_sc as plsc

_N = 10000
_E = 320000
_D = 128
_EPS = 1e-5
_CHUNK = 128
_NCHUNKS = _E // _CHUNK
_NTILES = 32
_ZP = 80
_DEG_W = 16

_NACC = _N
_ZPIECES = _NACC // _ZP
_DPIECES = _N // _ZP
_ZITERS = (_ZPIECES + 15) // 16
_DITERS = (_DPIECES + 15) // 16
_UNROLL = 3


def _sc_compiler_params():
    cp = pltpu.CompilerParams()
    if "needs_layout_passes" in pltpu.CompilerParams.__dataclass_fields__:
        cp = dataclasses.replace(cp, needs_layout_passes=False)
    return cp

def _sc_degree_body(col_hbm, out_hbm, counts, col_v):
    cid = lax.axis_index("c")
    sid = lax.axis_index("s")
    w = sid * 2 + cid

    zero16 = jnp.zeros((16,), jnp.float32)
    one16 = jnp.ones((16,), jnp.float32)

    @pl.loop(0, _N // 16)
    def _(i):
        counts[pl.ds(i * 16, 16)] = zero16

    @pl.loop(0, (_NCHUNKS + _NTILES - 1) // _NTILES)
    def _(kk):
        ch = kk * _NTILES + w

        @pl.when(ch < _NCHUNKS)
        def _():
            pltpu.sync_copy(col_hbm.at[pl.ds(ch * _CHUNK, _CHUNK)], col_v)
            for j in range(_CHUNK // 16):
                idx = col_v[pl.ds(j * 16, 16)]
                plsc.addupdate_scatter(counts, [idx], one16)

    pltpu.sync_copy(counts, out_hbm.at[w, 0])


def _sc_degree(col1d):
    mesh = plsc.VectorSubcoreMesh(core_axis_name="c", subcore_axis_name="s")
    kern = pl.kernel(
        _sc_degree_body,
        out_type=jax.ShapeDtypeStruct((_NTILES, 1, _N), jnp.float32),
        mesh=mesh,
        scratch_types=[
            pltpu.VMEM((_N,), jnp.float32),
            pltpu.VMEM((_CHUNK,), jnp.int32),
        ],
        compiler_params=_sc_compiler_params(),
    )
    return kern(col1d)


def _sc_propagate_body(h_hbm, rc_hbm, out_hbm, acc, rc_v, msgs, gsems,
                       ssems):
    cid = lax.axis_index("c")
    sid = lax.axis_index("s")
    w = sid * 2 + cid

    zero16 = jnp.zeros((16,), jnp.float32)

    @pl.loop(0, _ZP)
    def _(rr):
        for jj in range(_D // 16):
            msgs[0, rr, pl.ds(jj * 16, 16)] = zero16

    @pl.loop(0, _ZITERS)
    def _(kk):
        p = kk * 16 + sid

        @pl.when(p < _ZPIECES)
        def _():
            pltpu.sync_copy(msgs.at[0, pl.ds(0, _ZP)],
                            acc.at[pl.ds(p * _ZP, _ZP)])

    plsc.subcore_barrier()

    def _one(ch, b):
        pltpu.sync_copy(rc_hbm.at[ch], rc_v.at[b])
        return pltpu.async_copy(h_hbm.at[rc_v.at[b, 0]], msgs.at[b],
                                gsems.at[b])

    def _flush(b):
        pltpu.sync_copy(msgs.at[b], acc.at[rc_v.at[b, 1]], add=True)

    @pl.loop(0, _NCHUNKS // _NTILES // _UNROLL)
    def _(kg):
        base = (_UNROLL * kg) * _NTILES + w
        cps = [_one(base + b * _NTILES, b) for b in range(_UNROLL)]
        scs = []
        for b in range(_UNROLL):
            cps[b].wait()
            scs.append(pltpu.async_copy(msgs.at[b], acc.at[rc_v.at[b, 1]],
                                        ssems.at[b], add=True))
        for sc in scs:
            sc.wait()

    @pl.when(w < _NCHUNKS - (_NCHUNKS // _NTILES) * _NTILES)
    def _():
        ch = (_NCHUNKS // _NTILES) * _NTILES + w
        cp = _one(ch, 0)
        cp.wait()
        _flush(0)

    plsc.subcore_barrier()

    @pl.loop(0, _DITERS)
    def _(kk):
        p = kk * 16 + sid

        @pl.when(p < _DPIECES)
        def _():
            pltpu.sync_copy(acc.at[pl.ds(p * _ZP, _ZP)],
                            out_hbm.at[cid, pl.ds(p * _ZP, _ZP)])


def _sc_propagate(h, rc3d):
    mesh = plsc.VectorSubcoreMesh(core_axis_name="c", subcore_axis_name="s")
    kern = pl.kernel(
        _sc_propagate_body,
        out_type=jax.ShapeDtypeStruct((2, _N, _D), jnp.float32),
        mesh=mesh,
        scratch_types=[
            pltpu.VMEM_SHARED((_NACC, _D), jnp.float32),
            pltpu.VMEM((_UNROLL, 2, _CHUNK), jnp.int32),
            pltpu.VMEM((_UNROLL, _CHUNK, _D), jnp.float32),
            pltpu.SemaphoreType.DMA((_UNROLL,)),
            pltpu.SemaphoreType.DMA((_UNROLL,)),
        ],
        compiler_params=_sc_compiler_params(),
    )
    return kern(h, rc3d)


def _matmul_t(a, w_ref):
    return lax.dot_general(a, w_ref[...], (((1,), (1,)), ((), ())),
                           preferred_element_type=jnp.float32,
                           precision=lax.Precision.HIGHEST)


def _bn(h, g, b):
    m = jnp.mean(h, axis=0, keepdims=True)
    v = jnp.mean((h - m) ** 2, axis=0, keepdims=True)
    return (h - m) / jnp.sqrt(v + _EPS) * g + b


def _rsqrt_deg(degp):
    dp = degp[:, 0, :]
    ones = jnp.ones((_NTILES, 1), jnp.float32)
    deg = lax.dot_general(dp, ones, (((0,), (0,)), ((), ())),
                          preferred_element_type=jnp.float32,
                          precision=lax.Precision.HIGHEST)
    return jnp.where(deg > 0.0, lax.rsqrt(deg), 0.0)


def _stage_a_body(x_ref, w_ref, b_ref, g_ref, bb_ref, o_ref):
    h = _matmul_t(x_ref[...], w_ref) + b_ref[...]
    h = _bn(h, g_ref[...], bb_ref[...])
    o_ref[...] = jnp.maximum(h, 0.0)


def _scale_body(h_ref, degp_ref, o_ref):
    r = _rsqrt_deg(degp_ref[...])
    o_ref[...] = h_ref[...] * r


def _stage_c_body(sp_ref, degp_ref, h0_ref, w_ref, b_ref, g_ref, bb_ref,
                  o_ref):
    r = _rsqrt_deg(degp_ref[...])
    sp = sp_ref[...]
    s = (sp[0] + sp[1]) * r
    h = _matmul_t(s, w_ref) + b_ref[...]
    h = jnp.maximum(_bn(h, g_ref[...], bb_ref[...]), 0.0) + h0_ref[...]
    o_ref[...] = h * r


def _stage_d_body(sp_ref, degp_ref, h0_ref, w_ref, b_ref, g_ref, bb_ref,
                  wc_ref, bc_ref, o_ref):
    r = _rsqrt_deg(degp_ref[...])
    sp = sp_ref[...]
    s = (sp[0] + sp[1]) * r
    h = _matmul_t(s, w_ref) + b_ref[...]
    h = jnp.maximum(_bn(h, g_ref[...], bb_ref[...]), 0.0) + h0_ref[...]
    o_ref[...] = _matmul_t(h, wc_ref) + bc_ref[...]


def _f32_out(shape):
    return jax.ShapeDtypeStruct(shape, jnp.float32)


def kernel(x, edge_index, fc0_W, fc0_b, conv1_W, conv1_b, conv2_W, conv2_b,
           bn0_g, bn0_b, bn1_g, bn1_b, bn2_g, bn2_b, cls_W, cls_b):
    ei = edge_index.astype(jnp.int32)
    col1d = ei[1]
    rc3d = ei.reshape(2, _NCHUNKS, _CHUNK).transpose(1, 0, 2)

    b0 = fc0_b.reshape(1, _D)
    g0 = bn0_g.reshape(1, _D)
    bb0 = bn0_b.reshape(1, _D)
    b1 = conv1_b.reshape(1, _D)
    g1 = bn1_g.reshape(1, _D)
    bb1 = bn1_b.reshape(1, _D)
    b2 = conv2_b.reshape(1, _D)
    g2 = bn2_g.reshape(1, _D)
    bb2 = bn2_b.reshape(1, _D)
    bc = cls_b.reshape(1, _D)

    degp = _sc_degree(col1d)

    h0 = pl.pallas_call(_stage_a_body, out_shape=_f32_out((_N, _D)))(
        x, fc0_W, b0, g0, bb0)

    h0s = pl.pallas_call(_scale_body, out_shape=_f32_out((_N, _D)))(
        h0, degp)

    s1p = _sc_propagate(h0s, rc3d)

    h1s = pl.pallas_call(_stage_c_body, out_shape=_f32_out((_N, _D)))(
        s1p, degp, h0, conv1_W, b1, g1, bb1)

    s2p = _sc_propagate(h1s, rc3d)

    out = pl.pallas_call(_stage_d_body, out_shape=_f32_out((_N, _D)))(
        s2p, degp, h0, conv2_W, b2, g2, bb2, cls_W, bc)

    return out
